# Initial kernel scaffold; baseline (speedup 1.0000x reference)
import jax
import jax.numpy as jnp
from jax.experimental import pallas as pl


def kernel(x, coords, params):
    raise NotImplementedError("write your pallas kernel here")


# speedup vs baseline: 1.0212x; 1.0212x over previous
import functools

import jax
import jax.numpy as jnp
from jax.experimental import pallas as pl
from jax.experimental.pallas import tpu as pltpu

NUM_POINTS = [256, 64, 16, 4]
OUT_CH = [64, 128, 256, 512]
K = 16
N_PTS = 1024
B = 32


def _dense(x, W, b):
    return x @ W + b


def _knn(q, ref, k):
    d = (jnp.sum(q * q, -1)[:, :, None]
         - 2.0 * jnp.einsum('bmd,bnd->bmn', q, ref)
         + jnp.sum(ref * ref, -1)[:, None, :])
    return jax.lax.top_k(-d, k)[1]


def _gather(x, idx):
    return jax.vmap(lambda xb, ib: xb[ib])(x, idx)


def _pt0_body(h_ref, kf_ref, lc_ref,
              W1_ref, b1_ref, Wq_ref, bq_ref, Wk_ref, bk_ref, Wv_ref, bv_ref,
              Wp1_ref, bp1_ref, Wp2_ref, bp2_ref, Wg1_ref, bg1_ref,
              Wg2_ref, bg2_ref, W2_ref, b2_ref, out_ref,
              *, n, k, mid):
    h = h_ref[0]
    kf = kf_ref[0]
    lc = lc_ref[0]
    W1 = W1_ref[...]
    b1 = b1_ref[...]
    q = _dense(_dense(h, W1, b1), Wq_ref[...], bq_ref[...])
    kn = _dense(kf, W1, b1)
    kk = _dense(kn, Wk_ref[...], bk_ref[...])
    vv = _dense(kn, Wv_ref[...], bv_ref[...])
    pos = _dense(jax.nn.relu(_dense(lc, Wp1_ref[...], bp1_ref[...])),
                 Wp2_ref[...], bp2_ref[...])
    q3 = jnp.broadcast_to(q[:, None, :], (n, k, mid)).reshape(n * k, mid)
    a = q3 - kk + pos
    a = _dense(jax.nn.relu(_dense(a, Wg1_ref[...], bg1_ref[...])),
               Wg2_ref[...], bg2_ref[...])
    a3 = a.reshape(n, k, mid)
    a3 = a3 - jnp.max(a3, axis=1, keepdims=True)
    e = jnp.exp(a3)
    a3 = e / jnp.sum(e, axis=1, keepdims=True)
    vp = (vv + pos).reshape(n, k, mid)
    y = jnp.sum(a3 * vp, axis=1)
    y = _dense(y, W2_ref[...], b2_ref[...])
    out_ref[0] = jax.nn.relu(h + y)


def _pt_block_pallas(h, kf, lc, p):
    b, n, ch = h.shape
    k = kf.shape[2]
    mid = p['Wq'].shape[0]
    kf2 = kf.reshape(b, n * k, ch)
    lc2 = lc.reshape(b, n * k, 3)

    def wspec(w):
        return pl.BlockSpec(w.shape, lambda i: (0,) * w.ndim)

    wargs = [p['W1'], p['b1'], p['Wq'], p['bq'], p['Wk'], p['bk'],
             p['Wv'], p['bv'], p['Wp1'], p['bp1'], p['Wp2'], p['bp2'],
             p['Wg1'], p['bg1'], p['Wg2'], p['bg2'], p['W2'], p['b2']]
    return pl.pallas_call(
        functools.partial(_pt0_body, n=n, k=k, mid=mid),
        grid=(b,),
        in_specs=[
            pl.BlockSpec((1, n, ch), lambda i: (i, 0, 0)),
            pl.BlockSpec((1, n * k, ch), lambda i: (i, 0, 0)),
            pl.BlockSpec((1, n * k, 3), lambda i: (i, 0, 0)),
        ] + [wspec(w) for w in wargs],
        out_specs=pl.BlockSpec((1, n, ch), lambda i: (i, 0, 0)),
        out_shape=jax.ShapeDtypeStruct((b, n, ch), jnp.float32),
    )(h, kf2, lc2, *wargs)


def _fps_single(coords, M):
    d0 = jnp.sum((coords - coords[0]) ** 2, -1)
    idxs = jnp.zeros((M,), jnp.int32)

    def body(i, st):
        dists, idxs = st
        nxt = jnp.argmax(dists).astype(jnp.int32)
        idxs = idxs.at[i].set(nxt)
        d = jnp.sum((coords - coords[nxt]) ** 2, -1)
        return (jnp.minimum(dists, d), idxs)

    _, idxs = jax.lax.fori_loop(1, M, body, (d0, idxs))
    return idxs


def _transition_down(x, coords, M, k, W, b):
    idxs = jax.vmap(_fps_single, in_axes=(0, None))(coords, M)
    fps_coords = jax.vmap(lambda c, i: c[i])(coords, idxs)
    nb = _knn(fps_coords, coords, k)
    nb_x = _gather(x, nb)
    nb_c = _gather(coords, nb)
    local = nb_c - fps_coords[:, :, None, :]
    feats = jax.nn.relu(_dense(jnp.concatenate([nb_x, local], -1), W, b))
    return jnp.max(feats, axis=2), fps_coords


def kernel(x, coords, params):
    x = jnp.transpose(x, (0, 2, 1))
    h = jax.nn.relu(_dense(x, params['Win'], params['bin']))
    nb = _knn(coords, coords, K)
    knn_feats = _gather(h, nb)
    local = _gather(coords, nb) - coords[:, :, None, :]
    h = _pt_block_pallas(h, knn_feats, local, params['pt0'])
    c = coords
    for i, M in enumerate(NUM_POINTS):
        h, c = _transition_down(h, c, M, K,
                                params['td'][i]['W'], params['td'][i]['b'])
    h = jnp.mean(h, axis=1)
    h = jax.nn.relu(_dense(h, params['dec'][0]['W'], params['dec'][0]['b']))
    h = jax.nn.relu(_dense(h, params['dec'][1]['W'], params['dec'][1]['b']))
    return _dense(h, params['dec'][2]['W'], params['dec'][2]['b'])


# speedup vs baseline: 1.0888x; 1.0662x over previous
import functools

import jax
import jax.numpy as jnp
from jax import lax
from jax.experimental import pallas as pl
from jax.experimental.pallas import tpu as pltpu
from jax.experimental.pallas import tpu_sc as plsc

NUM_POINTS = [256, 64, 16, 4]
OUT_CH = [64, 128, 256, 512]
K = 16
N_PTS = 1024
B = 32


def _dense(x, W, b):
    return x @ W + b


def _knn(q, ref, k):
    d = (jnp.sum(q * q, -1)[:, :, None]
         - 2.0 * jnp.einsum('bmd,bnd->bmn', q, ref)
         + jnp.sum(ref * ref, -1)[:, None, :])
    return jax.lax.top_k(-d, k)[1]


def _gather(x, idx):
    return jax.vmap(lambda xb, ib: xb[ib])(x, idx)


def _pt0_body(h_ref, kf_ref, lc_ref,
              W1_ref, b1_ref, Wq_ref, bq_ref, Wk_ref, bk_ref, Wv_ref, bv_ref,
              Wp1_ref, bp1_ref, Wp2_ref, bp2_ref, Wg1_ref, bg1_ref,
              Wg2_ref, bg2_ref, W2_ref, b2_ref, out_ref,
              *, n, k, mid):
    h = h_ref[0]
    kf = kf_ref[0]
    lc = lc_ref[0]
    W1 = W1_ref[...]
    b1 = b1_ref[...]
    q = _dense(_dense(h, W1, b1), Wq_ref[...], bq_ref[...])
    kn = _dense(kf, W1, b1)
    kk = _dense(kn, Wk_ref[...], bk_ref[...])
    vv = _dense(kn, Wv_ref[...], bv_ref[...])
    pos = _dense(jax.nn.relu(_dense(lc, Wp1_ref[...], bp1_ref[...])),
                 Wp2_ref[...], bp2_ref[...])
    q3 = jnp.broadcast_to(q[:, None, :], (n, k, mid)).reshape(n * k, mid)
    a = q3 - kk + pos
    a = _dense(jax.nn.relu(_dense(a, Wg1_ref[...], bg1_ref[...])),
               Wg2_ref[...], bg2_ref[...])
    a3 = a.reshape(n, k, mid)
    a3 = a3 - jnp.max(a3, axis=1, keepdims=True)
    e = jnp.exp(a3)
    a3 = e / jnp.sum(e, axis=1, keepdims=True)
    vp = (vv + pos).reshape(n, k, mid)
    y = jnp.sum(a3 * vp, axis=1)
    y = _dense(y, W2_ref[...], b2_ref[...])
    out_ref[0] = jax.nn.relu(h + y)


def _pt_block_pallas(h, kf, lc, p):
    b, n, ch = h.shape
    k = kf.shape[2]
    mid = p['Wq'].shape[0]
    kf2 = kf.reshape(b, n * k, ch)
    lc2 = lc.reshape(b, n * k, 3)

    def wspec(w):
        return pl.BlockSpec(w.shape, lambda i: (0,) * w.ndim)

    wargs = [p['W1'], p['b1'], p['Wq'], p['bq'], p['Wk'], p['bk'],
             p['Wv'], p['bv'], p['Wp1'], p['bp1'], p['Wp2'], p['bp2'],
             p['Wg1'], p['bg1'], p['Wg2'], p['bg2'], p['W2'], p['b2']]
    return pl.pallas_call(
        functools.partial(_pt0_body, n=n, k=k, mid=mid),
        grid=(b,),
        in_specs=[
            pl.BlockSpec((1, n, ch), lambda i: (i, 0, 0)),
            pl.BlockSpec((1, n * k, ch), lambda i: (i, 0, 0)),
            pl.BlockSpec((1, n * k, 3), lambda i: (i, 0, 0)),
        ] + [wspec(w) for w in wargs],
        out_specs=pl.BlockSpec((1, n, ch), lambda i: (i, 0, 0)),
        out_shape=jax.ShapeDtypeStruct((b, n, ch), jnp.float32),
    )(h, kf2, lc2, *wargs)


_LANE = 16
_BIG_I32 = 2**31 - 1


def _sc_sweep(x_ref, y_ref, z_ref, d_ref, px, py, pz, nv, first):
    lane = lax.iota(jnp.int32, _LANE)
    bv = jnp.full((_LANE,), -jnp.inf, jnp.float32)
    bi = jnp.zeros((_LANE,), jnp.int32)
    bx = jnp.zeros((_LANE,), jnp.float32)
    by = jnp.zeros((_LANE,), jnp.float32)
    bz = jnp.zeros((_LANE,), jnp.float32)
    for j in range(nv):
        sl = pl.ds(j * _LANE, _LANE)
        xv = x_ref[sl]
        yv = y_ref[sl]
        zv = z_ref[sl]
        dx = xv - px
        dy = yv - py
        dz = zv - pz
        dnew = (dx * dx + dy * dy) + dz * dz
        if first:
            d = dnew
        else:
            d = jnp.minimum(d_ref[sl], dnew)
        d_ref[sl] = d
        upd = d > bv
        bv = jnp.where(upd, d, bv)
        bi = jnp.where(upd, lane + (j * _LANE), bi)
        bx = jnp.where(upd, xv, bx)
        by = jnp.where(upd, yv, by)
        bz = jnp.where(upd, zv, bz)
    return bv, bi, bx, by, bz


def _sc_fps_stage(x_ref, y_ref, z_ref, d_ref, idx_ref, n, m):
    nv = n // _LANE
    lane = lax.iota(jnp.int32, _LANE)
    sel0 = lane == 0
    x16 = x_ref[pl.ds(0, _LANE)]
    y16 = y_ref[pl.ds(0, _LANE)]
    z16 = z_ref[pl.ds(0, _LANE)]
    px = jnp.sum(jnp.where(sel0, x16, 0.0))
    py = jnp.sum(jnp.where(sel0, y16, 0.0))
    pz = jnp.sum(jnp.where(sel0, z16, 0.0))
    carry = _sc_sweep(x_ref, y_ref, z_ref, d_ref, px, py, pz, nv, first=True)
    acc = jnp.zeros((_LANE,), jnp.int32)

    def body(t, st):
        bv, bi, bx, by, bz, acc = st
        gm = jnp.max(bv)
        cand = jnp.where(bv == gm, bi, _BIG_I32)
        nxt = jnp.min(cand)
        sel = cand == nxt
        px = jnp.sum(jnp.where(sel, bx, 0.0))
        py = jnp.sum(jnp.where(sel, by, 0.0))
        pz = jnp.sum(jnp.where(sel, bz, 0.0))
        acc = jnp.where(lane == t, nxt, acc)
        bv, bi, bx, by, bz = _sc_sweep(x_ref, y_ref, z_ref, d_ref,
                                       px, py, pz, nv, first=False)
        return bv, bi, bx, by, bz, acc

    n_groups = (m + _LANE - 1) // _LANE
    for g in range(n_groups):
        lo = 1 if g == 0 else 0
        hi = min(_LANE, m - g * _LANE)
        st = lax.fori_loop(lo, hi, body, carry + (acc,))
        carry, acc = st[:5], st[5]
        idx_ref[pl.ds(g * _LANE, _LANE)] = acc


def _sc_gather_pts(src_x, src_y, src_z, idx_ref, dst_x, dst_y, dst_z, m):
    for t in range(m // _LANE):
        sl = pl.ds(t * _LANE, _LANE)
        iv = idx_ref[sl]
        dst_x[sl] = plsc.load_gather(src_x, [iv])
        dst_y[sl] = plsc.load_gather(src_y, [iv])
        dst_z[sl] = plsc.load_gather(src_z, [iv])


def _fps_sc_kernel(cx, cy, cz, idx1_o, idx2_o, idx3_o, idx4_o,
                   x0, y0, z0, d_s, i1_s, i2_s, i3_s, i4_s,
                   x1, y1, z1, x2, y2, z2, x3, y3, z3):
    wid = lax.axis_index("s") * 2 + lax.axis_index("c")
    pltpu.sync_copy(cx.at[wid], x0)
    pltpu.sync_copy(cy.at[wid], y0)
    pltpu.sync_copy(cz.at[wid], z0)
    _sc_fps_stage(x0, y0, z0, d_s, i1_s, N_PTS, NUM_POINTS[0])
    _sc_gather_pts(x0, y0, z0, i1_s, x1, y1, z1, NUM_POINTS[0])
    _sc_fps_stage(x1, y1, z1, d_s, i2_s, NUM_POINTS[0], NUM_POINTS[1])
    _sc_gather_pts(x1, y1, z1, i2_s, x2, y2, z2, NUM_POINTS[1])
    _sc_fps_stage(x2, y2, z2, d_s, i3_s, NUM_POINTS[1], NUM_POINTS[2])
    _sc_gather_pts(x2, y2, z2, i3_s, x3, y3, z3, NUM_POINTS[2])
    _sc_fps_stage(x3, y3, z3, d_s, i4_s, NUM_POINTS[2], NUM_POINTS[3])
    pltpu.sync_copy(i1_s, idx1_o.at[wid])
    pltpu.sync_copy(i2_s, idx2_o.at[wid])
    pltpu.sync_copy(i3_s, idx3_o.at[wid])
    pltpu.sync_copy(i4_s, idx4_o.at[wid])


def _fps_sc(coords):
    cx = coords[:, :, 0]
    cy = coords[:, :, 1]
    cz = coords[:, :, 2]
    i32 = jnp.int32
    f32 = jnp.float32
    mesh = plsc.VectorSubcoreMesh(core_axis_name="c", subcore_axis_name="s")
    fn = pl.kernel(
        _fps_sc_kernel,
        mesh=mesh,
        compiler_params=pltpu.CompilerParams(needs_layout_passes=False),
        out_type=[
            jax.ShapeDtypeStruct((B, 256), i32),
            jax.ShapeDtypeStruct((B, 64), i32),
            jax.ShapeDtypeStruct((B, 16), i32),
            jax.ShapeDtypeStruct((B, 16), i32),
        ],
        scratch_types=[
            pltpu.VMEM((N_PTS,), f32), pltpu.VMEM((N_PTS,), f32),
            pltpu.VMEM((N_PTS,), f32), pltpu.VMEM((N_PTS,), f32),
            pltpu.VMEM((256,), i32), pltpu.VMEM((64,), i32),
            pltpu.VMEM((16,), i32), pltpu.VMEM((16,), i32),
            pltpu.VMEM((256,), f32), pltpu.VMEM((256,), f32),
            pltpu.VMEM((256,), f32),
            pltpu.VMEM((64,), f32), pltpu.VMEM((64,), f32),
            pltpu.VMEM((64,), f32),
            pltpu.VMEM((16,), f32), pltpu.VMEM((16,), f32),
            pltpu.VMEM((16,), f32),
        ],
    )
    idx1, idx2, idx3, idx4 = fn(cx, cy, cz)
    return idx1, idx2, idx3, idx4[:, :4]


def _fps_single(coords, M):
    d0 = jnp.sum((coords - coords[0]) ** 2, -1)
    idxs = jnp.zeros((M,), jnp.int32)

    def body(i, st):
        dists, idxs = st
        nxt = jnp.argmax(dists).astype(jnp.int32)
        idxs = idxs.at[i].set(nxt)
        d = jnp.sum((coords - coords[nxt]) ** 2, -1)
        return (jnp.minimum(dists, d), idxs)

    _, idxs = jax.lax.fori_loop(1, M, body, (d0, idxs))
    return idxs


def _transition_down(x, coords, idxs, k, W, b):
    fps_coords = jax.vmap(lambda c, i: c[i])(coords, idxs)
    nb = _knn(fps_coords, coords, k)
    nb_x = _gather(x, nb)
    nb_c = _gather(coords, nb)
    local = nb_c - fps_coords[:, :, None, :]
    feats = jax.nn.relu(_dense(jnp.concatenate([nb_x, local], -1), W, b))
    return jnp.max(feats, axis=2), fps_coords


def kernel(x, coords, params):
    x = jnp.transpose(x, (0, 2, 1))
    h = jax.nn.relu(_dense(x, params['Win'], params['bin']))
    nb = _knn(coords, coords, K)
    knn_feats = _gather(h, nb)
    local = _gather(coords, nb) - coords[:, :, None, :]
    h = _pt_block_pallas(h, knn_feats, local, params['pt0'])
    fps_idxs = _fps_sc(coords)
    c = coords
    for i in range(len(NUM_POINTS)):
        h, c = _transition_down(h, c, fps_idxs[i], K,
                                params['td'][i]['W'], params['td'][i]['b'])
    h = jnp.mean(h, axis=1)
    h = jax.nn.relu(_dense(h, params['dec'][0]['W'], params['dec'][0]['b']))
    h = jax.nn.relu(_dense(h, params['dec'][1]['W'], params['dec'][1]['b']))
    return _dense(h, params['dec'][2]['W'], params['dec'][2]['b'])


# speedup vs baseline: 4.6981x; 4.3148x over previous
import functools

import jax
import jax.numpy as jnp
from jax import lax
from jax.experimental import pallas as pl
from jax.experimental.pallas import tpu as pltpu
from jax.experimental.pallas import tpu_sc as plsc

NUM_POINTS = [256, 64, 16, 4]
OUT_CH = [64, 128, 256, 512]
K = 16
N_PTS = 1024
B = 32


def _dense(x, W, b):
    return x @ W + b


def _knn(q, ref, k):
    d = (jnp.sum(q * q, -1)[:, :, None]
         - 2.0 * jnp.einsum('bmd,bnd->bmn', q, ref)
         + jnp.sum(ref * ref, -1)[:, None, :])
    return jax.lax.top_k(-d, k)[1]


def _gather(x, idx):
    return jax.vmap(lambda xb, ib: xb[ib])(x, idx)


def _pt0_body(h_ref, kf_ref, lc_ref,
              W1_ref, b1_ref, Wq_ref, bq_ref, Wk_ref, bk_ref, Wv_ref, bv_ref,
              Wp1_ref, bp1_ref, Wp2_ref, bp2_ref, Wg1_ref, bg1_ref,
              Wg2_ref, bg2_ref, W2_ref, b2_ref, out_ref,
              *, n, k, mid):
    h = h_ref[0]
    kf = kf_ref[0]
    lc = lc_ref[0]
    W1 = W1_ref[...]
    b1 = b1_ref[...]
    q = _dense(_dense(h, W1, b1), Wq_ref[...], bq_ref[...])
    kn = _dense(kf, W1, b1)
    kk = _dense(kn, Wk_ref[...], bk_ref[...])
    vv = _dense(kn, Wv_ref[...], bv_ref[...])
    pos = _dense(jax.nn.relu(_dense(lc, Wp1_ref[...], bp1_ref[...])),
                 Wp2_ref[...], bp2_ref[...])
    q3 = jnp.broadcast_to(q[:, None, :], (n, k, mid)).reshape(n * k, mid)
    a = q3 - kk + pos
    a = _dense(jax.nn.relu(_dense(a, Wg1_ref[...], bg1_ref[...])),
               Wg2_ref[...], bg2_ref[...])
    a3 = a.reshape(n, k, mid)
    a3 = a3 - jnp.max(a3, axis=1, keepdims=True)
    e = jnp.exp(a3)
    a3 = e / jnp.sum(e, axis=1, keepdims=True)
    vp = (vv + pos).reshape(n, k, mid)
    y = jnp.sum(a3 * vp, axis=1)
    y = _dense(y, W2_ref[...], b2_ref[...])
    out_ref[0] = jax.nn.relu(h + y)


def _pt_block_pallas(h, kf, lc, p):
    b, n, ch = h.shape
    k = kf.shape[2]
    mid = p['Wq'].shape[0]
    kf2 = kf.reshape(b, n * k, ch)
    lc2 = lc.reshape(b, n * k, 3)

    def wspec(w):
        return pl.BlockSpec(w.shape, lambda i: (0,) * w.ndim)

    wargs = [p['W1'], p['b1'], p['Wq'], p['bq'], p['Wk'], p['bk'],
             p['Wv'], p['bv'], p['Wp1'], p['bp1'], p['Wp2'], p['bp2'],
             p['Wg1'], p['bg1'], p['Wg2'], p['bg2'], p['W2'], p['b2']]
    return pl.pallas_call(
        functools.partial(_pt0_body, n=n, k=k, mid=mid),
        grid=(b,),
        in_specs=[
            pl.BlockSpec((1, n, ch), lambda i: (i, 0, 0)),
            pl.BlockSpec((1, n * k, ch), lambda i: (i, 0, 0)),
            pl.BlockSpec((1, n * k, 3), lambda i: (i, 0, 0)),
        ] + [wspec(w) for w in wargs],
        out_specs=pl.BlockSpec((1, n, ch), lambda i: (i, 0, 0)),
        out_shape=jax.ShapeDtypeStruct((b, n, ch), jnp.float32),
    )(h, kf2, lc2, *wargs)


_LANE = 16
_BIG_I32 = 2**31 - 1


def _sc_sweep(x_ref, y_ref, z_ref, d_ref, px, py, pz, nv, first):
    lane = lax.iota(jnp.int32, _LANE)
    bv = jnp.full((_LANE,), -jnp.inf, jnp.float32)
    bi = jnp.zeros((_LANE,), jnp.int32)
    bx = jnp.zeros((_LANE,), jnp.float32)
    by = jnp.zeros((_LANE,), jnp.float32)
    bz = jnp.zeros((_LANE,), jnp.float32)
    for j in range(nv):
        sl = pl.ds(j * _LANE, _LANE)
        xv = x_ref[sl]
        yv = y_ref[sl]
        zv = z_ref[sl]
        dx = xv - px
        dy = yv - py
        dz = zv - pz
        dnew = (dx * dx + dy * dy) + dz * dz
        if first:
            d = dnew
        else:
            d = jnp.minimum(d_ref[sl], dnew)
        d_ref[sl] = d
        upd = d > bv
        bv = jnp.where(upd, d, bv)
        bi = jnp.where(upd, lane + (j * _LANE), bi)
        bx = jnp.where(upd, xv, bx)
        by = jnp.where(upd, yv, by)
        bz = jnp.where(upd, zv, bz)
    return bv, bi, bx, by, bz


def _sc_fps_stage(x_ref, y_ref, z_ref, d_ref, idx_ref, n, m):
    nv = n // _LANE
    lane = lax.iota(jnp.int32, _LANE)
    sel0 = lane == 0
    x16 = x_ref[pl.ds(0, _LANE)]
    y16 = y_ref[pl.ds(0, _LANE)]
    z16 = z_ref[pl.ds(0, _LANE)]
    px = jnp.sum(jnp.where(sel0, x16, 0.0))
    py = jnp.sum(jnp.where(sel0, y16, 0.0))
    pz = jnp.sum(jnp.where(sel0, z16, 0.0))
    carry = _sc_sweep(x_ref, y_ref, z_ref, d_ref, px, py, pz, nv, first=True)
    acc = jnp.zeros((_LANE,), jnp.int32)

    def body(t, st):
        bv, bi, bx, by, bz, acc = st
        gm = jnp.max(bv)
        cand = jnp.where(bv == gm, bi, _BIG_I32)
        nxt = jnp.min(cand)
        sel = cand == nxt
        px = jnp.sum(jnp.where(sel, bx, 0.0))
        py = jnp.sum(jnp.where(sel, by, 0.0))
        pz = jnp.sum(jnp.where(sel, bz, 0.0))
        acc = jnp.where(lane == t, nxt, acc)
        bv, bi, bx, by, bz = _sc_sweep(x_ref, y_ref, z_ref, d_ref,
                                       px, py, pz, nv, first=False)
        return bv, bi, bx, by, bz, acc

    n_groups = (m + _LANE - 1) // _LANE
    for g in range(n_groups):
        lo = 1 if g == 0 else 0
        hi = min(_LANE, m - g * _LANE)
        st = lax.fori_loop(lo, hi, body, carry + (acc,))
        carry, acc = st[:5], st[5]
        idx_ref[pl.ds(g * _LANE, _LANE)] = acc


def _sc_gather_pts(src_x, src_y, src_z, idx_ref, dst_x, dst_y, dst_z, m):
    for t in range(m // _LANE):
        sl = pl.ds(t * _LANE, _LANE)
        iv = idx_ref[sl]
        dst_x[sl] = plsc.load_gather(src_x, [iv])
        dst_y[sl] = plsc.load_gather(src_y, [iv])
        dst_z[sl] = plsc.load_gather(src_z, [iv])


def _fps_sc_kernel(cx, cy, cz, idx1_o, idx2_o, idx3_o, idx4_o,
                   c1_o, c2_o, c3_o,
                   x0, y0, z0, d_s, i1_s, i2_s, i3_s, i4_s,
                   x1, y1, z1, x2, y2, z2, x3, y3, z3):
    wid = lax.axis_index("s") * 2 + lax.axis_index("c")
    pltpu.sync_copy(cx.at[wid], x0)
    pltpu.sync_copy(cy.at[wid], y0)
    pltpu.sync_copy(cz.at[wid], z0)
    _sc_fps_stage(x0, y0, z0, d_s, i1_s, N_PTS, NUM_POINTS[0])
    _sc_gather_pts(x0, y0, z0, i1_s, x1, y1, z1, NUM_POINTS[0])
    _sc_fps_stage(x1, y1, z1, d_s, i2_s, NUM_POINTS[0], NUM_POINTS[1])
    _sc_gather_pts(x1, y1, z1, i2_s, x2, y2, z2, NUM_POINTS[1])
    _sc_fps_stage(x2, y2, z2, d_s, i3_s, NUM_POINTS[1], NUM_POINTS[2])
    _sc_gather_pts(x2, y2, z2, i3_s, x3, y3, z3, NUM_POINTS[2])
    _sc_fps_stage(x3, y3, z3, d_s, i4_s, NUM_POINTS[2], NUM_POINTS[3])
    pltpu.sync_copy(i1_s, idx1_o.at[wid])
    pltpu.sync_copy(i2_s, idx2_o.at[wid])
    pltpu.sync_copy(i3_s, idx3_o.at[wid])
    pltpu.sync_copy(i4_s, idx4_o.at[wid])
    pltpu.sync_copy(x1, c1_o.at[wid * 3 + 0])
    pltpu.sync_copy(y1, c1_o.at[wid * 3 + 1])
    pltpu.sync_copy(z1, c1_o.at[wid * 3 + 2])
    pltpu.sync_copy(x2, c2_o.at[wid * 3 + 0])
    pltpu.sync_copy(y2, c2_o.at[wid * 3 + 1])
    pltpu.sync_copy(z2, c2_o.at[wid * 3 + 2])
    pltpu.sync_copy(x3, c3_o.at[wid * 3 + 0])
    pltpu.sync_copy(y3, c3_o.at[wid * 3 + 1])
    pltpu.sync_copy(z3, c3_o.at[wid * 3 + 2])


def _fps_sc(coords):
    cx = coords[:, :, 0]
    cy = coords[:, :, 1]
    cz = coords[:, :, 2]
    i32 = jnp.int32
    f32 = jnp.float32
    mesh = plsc.VectorSubcoreMesh(core_axis_name="c", subcore_axis_name="s")
    fn = pl.kernel(
        _fps_sc_kernel,
        mesh=mesh,
        compiler_params=pltpu.CompilerParams(needs_layout_passes=False, use_tc_tiling_on_sc=False),
        out_type=[
            jax.ShapeDtypeStruct((B, 256), i32),
            jax.ShapeDtypeStruct((B, 64), i32),
            jax.ShapeDtypeStruct((B, 16), i32),
            jax.ShapeDtypeStruct((B, 16), i32),
            jax.ShapeDtypeStruct((B * 3, 256), f32),
            jax.ShapeDtypeStruct((B * 3, 64), f32),
            jax.ShapeDtypeStruct((B * 3, 16), f32),
        ],
        scratch_types=[
            pltpu.VMEM((N_PTS,), f32), pltpu.VMEM((N_PTS,), f32),
            pltpu.VMEM((N_PTS,), f32), pltpu.VMEM((N_PTS,), f32),
            pltpu.VMEM((256,), i32), pltpu.VMEM((64,), i32),
            pltpu.VMEM((16,), i32), pltpu.VMEM((16,), i32),
            pltpu.VMEM((256,), f32), pltpu.VMEM((256,), f32),
            pltpu.VMEM((256,), f32),
            pltpu.VMEM((64,), f32), pltpu.VMEM((64,), f32),
            pltpu.VMEM((64,), f32),
            pltpu.VMEM((16,), f32), pltpu.VMEM((16,), f32),
            pltpu.VMEM((16,), f32),
        ],
    )
    idx1, idx2, idx3, idx4, c1o, c2o, c3o = fn(cx, cy, cz)
    return ((idx1, idx2, idx3, idx4[:, :4]),
            (c1o.reshape(B, 3, 256), c2o.reshape(B, 3, 64),
             c3o.reshape(B, 3, 16)))


def _make_group_sc_kernel(M, N, ch, chunk, nchunks):
    def body(tab, nbg_h, qx_h, qy_h, qz_h, nx_h, ny_h, nz_h,
             outf, outl,
             nbg, qx, qy, qz, nx, ny, nz, buf0, buf1, lcl, sem0, sem1):
        wid = lax.axis_index("s") * 2 + lax.axis_index("c")
        pltpu.sync_copy(nbg_h.at[wid], nbg)
        pltpu.sync_copy(qx_h.at[wid], qx)
        pltpu.sync_copy(qy_h.at[wid], qy)
        pltpu.sync_copy(qz_h.at[wid], qz)
        pltpu.sync_copy(nx_h.at[wid], nx)
        pltpu.sync_copy(ny_h.at[wid], ny)
        pltpu.sync_copy(nz_h.at[wid], nz)
        bufs = (buf0, buf1)
        sems = (sem0, sem1)
        base = wid * (M * K)
        prev = None
        for j in range(nchunks):
            cur = pltpu.async_copy(
                tab.at[nbg.at[pl.ds(j * chunk, chunk)]],
                bufs[j % 2], sems[j % 2])
            if prev is not None:
                prev.wait()
                pltpu.sync_copy(bufs[(j - 1) % 2],
                                outf.at[pl.ds(base + (j - 1) * chunk, chunk)])
            prev = cur
        prev.wait()
        pltpu.sync_copy(bufs[(nchunks - 1) % 2],
                        outf.at[pl.ds(base + (nchunks - 1) * chunk, chunk)])

        lane = lax.iota(jnp.int32, _LANE)
        off = jnp.full((_LANE,), wid * N, jnp.int32)

        def lbody(m, carry):
            ivg = plsc.load_gather(nbg, [jnp.full((_LANE,), m * _LANE,
                                                  jnp.int32) + lane])
            iv = ivg - off
            mfull = jnp.full((_LANE,), m, jnp.int32)
            gx = plsc.load_gather(nx, [iv]) - plsc.load_gather(qx, [mfull])
            gy = plsc.load_gather(ny, [iv]) - plsc.load_gather(qy, [mfull])
            gz = plsc.load_gather(nz, [iv]) - plsc.load_gather(qz, [mfull])
            lbase = jnp.full((_LANE,), m * 48, jnp.int32) + lane * 3
            plsc.store_scatter(lcl, [lbase], gx)
            plsc.store_scatter(lcl, [lbase + 1], gy)
            plsc.store_scatter(lcl, [lbase + 2], gz)
            return carry

        lax.fori_loop(0, M, lbody, 0)
        pltpu.sync_copy(lcl, outl.at[wid])

    return body


def _group_sc(tab, nbg, qpl, npl, M, N, ch):
    i32 = jnp.int32
    f32 = jnp.float32
    MK = M * K
    chunk = min(128, MK)
    nchunks = MK // chunk
    Mp = qpl[0].shape[1]
    mesh = plsc.VectorSubcoreMesh(core_axis_name="c", subcore_axis_name="s")
    fn = pl.kernel(
        _make_group_sc_kernel(M, N, ch, chunk, nchunks),
        mesh=mesh,
        compiler_params=pltpu.CompilerParams(needs_layout_passes=False, use_tc_tiling_on_sc=False),
        out_type=[
            jax.ShapeDtypeStruct((B * MK, ch), f32),
            jax.ShapeDtypeStruct((B, M * 48), f32),
        ],
        scratch_types=[
            pltpu.VMEM((MK,), i32),
            pltpu.VMEM((Mp,), f32), pltpu.VMEM((Mp,), f32),
            pltpu.VMEM((Mp,), f32),
            pltpu.VMEM((N,), f32), pltpu.VMEM((N,), f32),
            pltpu.VMEM((N,), f32),
            pltpu.VMEM((chunk, ch), f32), pltpu.VMEM((chunk, ch), f32),
            pltpu.VMEM((M * 48,), f32),
            pltpu.SemaphoreType.DMA, pltpu.SemaphoreType.DMA,
        ],
    )
    feats, local = fn(tab, nbg, qpl[0], qpl[1], qpl[2],
                      npl[0], npl[1], npl[2])
    return feats.reshape(B, M, K, ch), local.reshape(B, M, K, 3)


def _global_idx(nb, N):
    return (nb + (jnp.arange(B, dtype=jnp.int32) * N)[:, None, None]
            ).reshape(B, -1)


def _pad_planes(pl3, width):
    if pl3.shape[2] == width:
        return pl3
    pad = width - pl3.shape[2]
    return jnp.pad(pl3, ((0, 0), (0, 0), (0, pad)))


def _transition_down(nb_x, local, W, b):
    feats = jax.nn.relu(_dense(jnp.concatenate([nb_x, local], -1), W, b))
    return jnp.max(feats, axis=2)


def kernel(x, coords, params):
    x = jnp.transpose(x, (0, 2, 1))
    h = jax.nn.relu(_dense(x, params['Win'], params['bin']))
    c0pl = (coords[:, :, 0], coords[:, :, 1], coords[:, :, 2])
    nb0 = _knn(coords, coords, K)
    kf0, lc0 = _group_sc(h.reshape(B * N_PTS, -1), _global_idx(nb0, N_PTS),
                         c0pl, c0pl, N_PTS, N_PTS, h.shape[-1])
    h = _pt_block_pallas(h, kf0, lc0, params['pt0'])
    fps_idxs, fps_cpl = _fps_sc(coords)
    cs = [coords] + [cpl.transpose(0, 2, 1) for cpl in fps_cpl]
    c4 = jax.vmap(lambda c, i: c[i])(cs[3], fps_idxs[3])
    cs.append(c4)
    c4pl = (c4[:, :, 0], c4[:, :, 1], c4[:, :, 2])
    plns = [c0pl] + [tuple(p[:, i_, :] for i_ in range(3)) for p in fps_cpl]
    plns.append(tuple(jnp.pad(p, ((0, 0), (0, 12))) for p in c4pl))
    sizes = [N_PTS] + NUM_POINTS
    for i in range(len(NUM_POINTS)):
        M, N = sizes[i + 1], sizes[i]
        nb = _knn(cs[i + 1], cs[i], K)
        nb_x, local = _group_sc(h.reshape(B * N, -1), _global_idx(nb, N),
                                plns[i + 1], plns[i], M, N, h.shape[-1])
        h = _transition_down(nb_x, local,
                             params['td'][i]['W'], params['td'][i]['b'])
    h = jnp.mean(h, axis=1)
    h = jax.nn.relu(_dense(h, params['dec'][0]['W'], params['dec'][0]['b']))
    h = jax.nn.relu(_dense(h, params['dec'][1]['W'], params['dec'][1]['b']))
    return _dense(h, params['dec'][2]['W'], params['dec'][2]['b'])


# speedup vs baseline: 11.2134x; 2.3868x over previous
import functools

import jax
import jax.numpy as jnp
from jax import lax
from jax.experimental import pallas as pl
from jax.experimental.pallas import tpu as pltpu
from jax.experimental.pallas import tpu_sc as plsc

NUM_POINTS = [256, 64, 16, 4]
OUT_CH = [64, 128, 256, 512]
K = 16
N_PTS = 1024
B = 32


def _dense(x, W, b):
    return x @ W + b


def _knn(q, ref, k):
    d = (jnp.sum(q * q, -1)[:, :, None]
         - 2.0 * jnp.einsum('bmd,bnd->bmn', q, ref)
         + jnp.sum(ref * ref, -1)[:, None, :])
    return jax.lax.top_k(-d, k)[1]


def _gather(x, idx):
    return jax.vmap(lambda xb, ib: xb[ib])(x, idx)


def _pt0_body(h_ref, kf_ref, lc_ref,
              W1_ref, b1_ref, Wq_ref, bq_ref, Wk_ref, bk_ref, Wv_ref, bv_ref,
              Wp1_ref, bp1_ref, Wp2_ref, bp2_ref, Wg1_ref, bg1_ref,
              Wg2_ref, bg2_ref, W2_ref, b2_ref, out_ref,
              *, n, k, mid):
    h = h_ref[0]
    kf = kf_ref[0]
    lc = lc_ref[0]
    W1 = W1_ref[...]
    b1 = b1_ref[...]
    q = _dense(_dense(h, W1, b1), Wq_ref[...], bq_ref[...])
    kn = _dense(kf, W1, b1)
    kk = _dense(kn, Wk_ref[...], bk_ref[...])
    vv = _dense(kn, Wv_ref[...], bv_ref[...])
    pos = _dense(jax.nn.relu(_dense(lc, Wp1_ref[...], bp1_ref[...])),
                 Wp2_ref[...], bp2_ref[...])
    q3 = jnp.broadcast_to(q[:, None, :], (n, k, mid)).reshape(n * k, mid)
    a = q3 - kk + pos
    a = _dense(jax.nn.relu(_dense(a, Wg1_ref[...], bg1_ref[...])),
               Wg2_ref[...], bg2_ref[...])
    a3 = a.reshape(n, k, mid)
    a3 = a3 - jnp.max(a3, axis=1, keepdims=True)
    e = jnp.exp(a3)
    a3 = e / jnp.sum(e, axis=1, keepdims=True)
    vp = (vv + pos).reshape(n, k, mid)
    y = jnp.sum(a3 * vp, axis=1)
    y = _dense(y, W2_ref[...], b2_ref[...])
    out_ref[0] = jax.nn.relu(h + y)


def _pt_block_pallas(h, kf, lc, p):
    b, n, ch = h.shape
    k = kf.shape[2]
    mid = p['Wq'].shape[0]
    kf2 = kf.reshape(b, n * k, ch)
    lc2 = lc.reshape(b, n * k, 3)

    def wspec(w):
        return pl.BlockSpec(w.shape, lambda i: (0,) * w.ndim)

    wargs = [p['W1'], p['b1'], p['Wq'], p['bq'], p['Wk'], p['bk'],
             p['Wv'], p['bv'], p['Wp1'], p['bp1'], p['Wp2'], p['bp2'],
             p['Wg1'], p['bg1'], p['Wg2'], p['bg2'], p['W2'], p['b2']]
    return pl.pallas_call(
        functools.partial(_pt0_body, n=n, k=k, mid=mid),
        grid=(b,),
        in_specs=[
            pl.BlockSpec((1, n, ch), lambda i: (i, 0, 0)),
            pl.BlockSpec((1, n * k, ch), lambda i: (i, 0, 0)),
            pl.BlockSpec((1, n * k, 3), lambda i: (i, 0, 0)),
        ] + [wspec(w) for w in wargs],
        out_specs=pl.BlockSpec((1, n, ch), lambda i: (i, 0, 0)),
        out_shape=jax.ShapeDtypeStruct((b, n, ch), jnp.float32),
    )(h, kf2, lc2, *wargs)


_LANE = 16
_BIG_I32 = 2**31 - 1


def _sc_sweep(x_ref, y_ref, z_ref, d_ref, px, py, pz, nv, first):
    lane = lax.iota(jnp.int32, _LANE)
    bv = jnp.full((_LANE,), -jnp.inf, jnp.float32)
    bi = jnp.zeros((_LANE,), jnp.int32)
    bx = jnp.zeros((_LANE,), jnp.float32)
    by = jnp.zeros((_LANE,), jnp.float32)
    bz = jnp.zeros((_LANE,), jnp.float32)
    for j in range(nv):
        sl = pl.ds(j * _LANE, _LANE)
        xv = x_ref[sl]
        yv = y_ref[sl]
        zv = z_ref[sl]
        dx = xv - px
        dy = yv - py
        dz = zv - pz
        dnew = (dx * dx + dy * dy) + dz * dz
        if first:
            d = dnew
        else:
            d = jnp.minimum(d_ref[sl], dnew)
        d_ref[sl] = d
        upd = d > bv
        bv = jnp.where(upd, d, bv)
        bi = jnp.where(upd, lane + (j * _LANE), bi)
        bx = jnp.where(upd, xv, bx)
        by = jnp.where(upd, yv, by)
        bz = jnp.where(upd, zv, bz)
    return bv, bi, bx, by, bz


def _sc_fps_stage(x_ref, y_ref, z_ref, d_ref, idx_ref, n, m):
    nv = n // _LANE
    lane = lax.iota(jnp.int32, _LANE)
    sel0 = lane == 0
    x16 = x_ref[pl.ds(0, _LANE)]
    y16 = y_ref[pl.ds(0, _LANE)]
    z16 = z_ref[pl.ds(0, _LANE)]
    px = jnp.sum(jnp.where(sel0, x16, 0.0))
    py = jnp.sum(jnp.where(sel0, y16, 0.0))
    pz = jnp.sum(jnp.where(sel0, z16, 0.0))
    carry = _sc_sweep(x_ref, y_ref, z_ref, d_ref, px, py, pz, nv, first=True)
    acc = jnp.zeros((_LANE,), jnp.int32)

    def body(t, st):
        bv, bi, bx, by, bz, acc = st
        gm = jnp.max(bv)
        cand = jnp.where(bv == gm, bi, _BIG_I32)
        nxt = jnp.min(cand)
        sel = cand == nxt
        px = jnp.sum(jnp.where(sel, bx, 0.0))
        py = jnp.sum(jnp.where(sel, by, 0.0))
        pz = jnp.sum(jnp.where(sel, bz, 0.0))
        acc = jnp.where(lane == t, nxt, acc)
        bv, bi, bx, by, bz = _sc_sweep(x_ref, y_ref, z_ref, d_ref,
                                       px, py, pz, nv, first=False)
        return bv, bi, bx, by, bz, acc

    n_groups = (m + _LANE - 1) // _LANE
    for g in range(n_groups):
        lo = 1 if g == 0 else 0
        hi = min(_LANE, m - g * _LANE)
        st = lax.fori_loop(lo, hi, body, carry + (acc,))
        carry, acc = st[:5], st[5]
        idx_ref[pl.ds(g * _LANE, _LANE)] = acc


def _sc_gather_pts(src_x, src_y, src_z, idx_ref, dst_x, dst_y, dst_z, m):
    for t in range(m // _LANE):
        sl = pl.ds(t * _LANE, _LANE)
        iv = idx_ref[sl]
        dst_x[sl] = plsc.load_gather(src_x, [iv])
        dst_y[sl] = plsc.load_gather(src_y, [iv])
        dst_z[sl] = plsc.load_gather(src_z, [iv])


def _fps_sc_kernel(cx, cy, cz, idx1_o, idx2_o, idx3_o, idx4_o,
                   c1_o, c2_o, c3_o,
                   x0, y0, z0, d_s, i1_s, i2_s, i3_s, i4_s,
                   x1, y1, z1, x2, y2, z2, x3, y3, z3):
    wid = lax.axis_index("s") * 2 + lax.axis_index("c")
    pltpu.sync_copy(cx.at[wid], x0)
    pltpu.sync_copy(cy.at[wid], y0)
    pltpu.sync_copy(cz.at[wid], z0)
    _sc_fps_stage(x0, y0, z0, d_s, i1_s, N_PTS, NUM_POINTS[0])
    _sc_gather_pts(x0, y0, z0, i1_s, x1, y1, z1, NUM_POINTS[0])
    _sc_fps_stage(x1, y1, z1, d_s, i2_s, NUM_POINTS[0], NUM_POINTS[1])
    _sc_gather_pts(x1, y1, z1, i2_s, x2, y2, z2, NUM_POINTS[1])
    _sc_fps_stage(x2, y2, z2, d_s, i3_s, NUM_POINTS[1], NUM_POINTS[2])
    _sc_gather_pts(x2, y2, z2, i3_s, x3, y3, z3, NUM_POINTS[2])
    _sc_fps_stage(x3, y3, z3, d_s, i4_s, NUM_POINTS[2], NUM_POINTS[3])
    pltpu.sync_copy(i1_s, idx1_o.at[wid])
    pltpu.sync_copy(i2_s, idx2_o.at[wid])
    pltpu.sync_copy(i3_s, idx3_o.at[wid])
    pltpu.sync_copy(i4_s, idx4_o.at[wid])
    pltpu.sync_copy(x1, c1_o.at[wid * 3 + 0])
    pltpu.sync_copy(y1, c1_o.at[wid * 3 + 1])
    pltpu.sync_copy(z1, c1_o.at[wid * 3 + 2])
    pltpu.sync_copy(x2, c2_o.at[wid * 3 + 0])
    pltpu.sync_copy(y2, c2_o.at[wid * 3 + 1])
    pltpu.sync_copy(z2, c2_o.at[wid * 3 + 2])
    pltpu.sync_copy(x3, c3_o.at[wid * 3 + 0])
    pltpu.sync_copy(y3, c3_o.at[wid * 3 + 1])
    pltpu.sync_copy(z3, c3_o.at[wid * 3 + 2])


def _fps_sc(coords):
    cx = coords[:, :, 0]
    cy = coords[:, :, 1]
    cz = coords[:, :, 2]
    i32 = jnp.int32
    f32 = jnp.float32
    mesh = plsc.VectorSubcoreMesh(core_axis_name="c", subcore_axis_name="s")
    fn = pl.kernel(
        _fps_sc_kernel,
        mesh=mesh,
        compiler_params=pltpu.CompilerParams(needs_layout_passes=False, use_tc_tiling_on_sc=False),
        out_type=[
            jax.ShapeDtypeStruct((B, 256), i32),
            jax.ShapeDtypeStruct((B, 64), i32),
            jax.ShapeDtypeStruct((B, 16), i32),
            jax.ShapeDtypeStruct((B, 16), i32),
            jax.ShapeDtypeStruct((B * 3, 256), f32),
            jax.ShapeDtypeStruct((B * 3, 64), f32),
            jax.ShapeDtypeStruct((B * 3, 16), f32),
        ],
        scratch_types=[
            pltpu.VMEM((N_PTS,), f32), pltpu.VMEM((N_PTS,), f32),
            pltpu.VMEM((N_PTS,), f32), pltpu.VMEM((N_PTS,), f32),
            pltpu.VMEM((256,), i32), pltpu.VMEM((64,), i32),
            pltpu.VMEM((16,), i32), pltpu.VMEM((16,), i32),
            pltpu.VMEM((256,), f32), pltpu.VMEM((256,), f32),
            pltpu.VMEM((256,), f32),
            pltpu.VMEM((64,), f32), pltpu.VMEM((64,), f32),
            pltpu.VMEM((64,), f32),
            pltpu.VMEM((16,), f32), pltpu.VMEM((16,), f32),
            pltpu.VMEM((16,), f32),
        ],
    )
    idx1, idx2, idx3, idx4, c1o, c2o, c3o = fn(cx, cy, cz)
    return ((idx1, idx2, idx3, idx4[:, :4]),
            (c1o.reshape(B, 3, 256), c2o.reshape(B, 3, 64),
             c3o.reshape(B, 3, 16)))


def _make_knn_sc_kernel(M, N, CH):
    nc = N // _LANE

    def body(d_h, nb_o, dbuf, nbuf, sem0):
        wid = lax.axis_index("s") * 2 + lax.axis_index("c")
        lane = lax.iota(jnp.int32, _LANE)
        base = wid * (M * N)

        def row_top16(r):
            tv = ti = None
            for j in range(nc):
                sl = pl.ds(r * N + j * _LANE, _LANE)
                d = dbuf[sl]
                iv = lane + (j * _LANE)
                if tv is None:
                    tv, ti = plsc.sort_key_val(d, iv)
                else:
                    ca, cia = plsc.sort_key_val(d, iv)
                    cv = lax.rev(ca, (0,))
                    ci = lax.rev(cia, (0,))
                    sel = cv < tv
                    mk = jnp.where(sel, cv, tv)
                    mv = jnp.where(sel, ci, ti)
                    tv, ti = plsc.sort_key_val(mk, mv)
            return ti

        def gbody(g, carry):
            pltpu.async_copy(
                d_h.at[pl.ds(base + g * (CH * N), CH * N)], dbuf, sem0
            ).wait()
            for r in range(CH):
                ti = row_top16(r)
                plsc.store_scatter(
                    nbuf, [(g * (CH * _LANE) + r * _LANE) + lane], ti)
            return carry

        lax.fori_loop(0, M // CH, gbody, 0)
        pltpu.sync_copy(nbuf, nb_o.at[wid])

    return body


def _knn_sc(d, M, N):
    CH = min(4, M)
    mesh = plsc.VectorSubcoreMesh(core_axis_name="c", subcore_axis_name="s")
    fn = pl.kernel(
        _make_knn_sc_kernel(M, N, CH),
        mesh=mesh,
        compiler_params=pltpu.CompilerParams(needs_layout_passes=False,
                                             use_tc_tiling_on_sc=False),
        out_type=[jax.ShapeDtypeStruct((B, M * _LANE), jnp.int32)],
        scratch_types=[
            pltpu.VMEM((CH * N,), jnp.float32),
            pltpu.VMEM((M * _LANE,), jnp.int32),
            pltpu.SemaphoreType.DMA,
        ],
    )
    nb, = fn(d.reshape(B * M * N))
    return nb.reshape(B, M, _LANE)


def _dist(q, ref):
    return (jnp.sum(q * q, -1)[:, :, None]
            - 2.0 * jnp.einsum('bmd,bnd->bmn', q, ref)
            + jnp.sum(ref * ref, -1)[:, None, :])


def _make_group_sc_kernel(M, N, ch, chunk, nchunks):
    def body(tab, nbg_h, qx_h, qy_h, qz_h, nx_h, ny_h, nz_h,
             outf, outl,
             nbg, qx, qy, qz, nx, ny, nz, buf0, buf1, lcl, sem0, sem1):
        wid = lax.axis_index("s") * 2 + lax.axis_index("c")
        pltpu.sync_copy(nbg_h.at[wid], nbg)
        pltpu.sync_copy(qx_h.at[wid], qx)
        pltpu.sync_copy(qy_h.at[wid], qy)
        pltpu.sync_copy(qz_h.at[wid], qz)
        pltpu.sync_copy(nx_h.at[wid], nx)
        pltpu.sync_copy(ny_h.at[wid], ny)
        pltpu.sync_copy(nz_h.at[wid], nz)
        bufs = (buf0, buf1)
        sems = (sem0, sem1)
        base = wid * (M * K)
        prev = None
        for j in range(nchunks):
            cur = pltpu.async_copy(
                tab.at[nbg.at[pl.ds(j * chunk, chunk)]],
                bufs[j % 2], sems[j % 2])
            if prev is not None:
                prev.wait()
                pltpu.sync_copy(bufs[(j - 1) % 2],
                                outf.at[pl.ds(base + (j - 1) * chunk, chunk)])
            prev = cur
        prev.wait()
        pltpu.sync_copy(bufs[(nchunks - 1) % 2],
                        outf.at[pl.ds(base + (nchunks - 1) * chunk, chunk)])

        lane = lax.iota(jnp.int32, _LANE)
        off = jnp.full((_LANE,), wid * N, jnp.int32)

        def lbody(m, carry):
            ivg = plsc.load_gather(nbg, [jnp.full((_LANE,), m * _LANE,
                                                  jnp.int32) + lane])
            iv = ivg - off
            mfull = jnp.full((_LANE,), m, jnp.int32)
            gx = plsc.load_gather(nx, [iv]) - plsc.load_gather(qx, [mfull])
            gy = plsc.load_gather(ny, [iv]) - plsc.load_gather(qy, [mfull])
            gz = plsc.load_gather(nz, [iv]) - plsc.load_gather(qz, [mfull])
            lbase = jnp.full((_LANE,), m * 48, jnp.int32) + lane * 3
            plsc.store_scatter(lcl, [lbase], gx)
            plsc.store_scatter(lcl, [lbase + 1], gy)
            plsc.store_scatter(lcl, [lbase + 2], gz)
            return carry

        lax.fori_loop(0, M, lbody, 0)
        pltpu.sync_copy(lcl, outl.at[wid])

    return body


def _group_sc(tab, nbg, qpl, npl, M, N, ch):
    i32 = jnp.int32
    f32 = jnp.float32
    MK = M * K
    chunk = min(128, MK)
    nchunks = MK // chunk
    Mp = qpl[0].shape[1]
    mesh = plsc.VectorSubcoreMesh(core_axis_name="c", subcore_axis_name="s")
    fn = pl.kernel(
        _make_group_sc_kernel(M, N, ch, chunk, nchunks),
        mesh=mesh,
        compiler_params=pltpu.CompilerParams(needs_layout_passes=False, use_tc_tiling_on_sc=False),
        out_type=[
            jax.ShapeDtypeStruct((B * MK, ch), f32),
            jax.ShapeDtypeStruct((B, M * 48), f32),
        ],
        scratch_types=[
            pltpu.VMEM((MK,), i32),
            pltpu.VMEM((Mp,), f32), pltpu.VMEM((Mp,), f32),
            pltpu.VMEM((Mp,), f32),
            pltpu.VMEM((N,), f32), pltpu.VMEM((N,), f32),
            pltpu.VMEM((N,), f32),
            pltpu.VMEM((chunk, ch), f32), pltpu.VMEM((chunk, ch), f32),
            pltpu.VMEM((M * 48,), f32),
            pltpu.SemaphoreType.DMA, pltpu.SemaphoreType.DMA,
        ],
    )
    feats, local = fn(tab, nbg, qpl[0], qpl[1], qpl[2],
                      npl[0], npl[1], npl[2])
    return feats.reshape(B, M, K, ch), local.reshape(B, M, K, 3)


def _global_idx(nb, N):
    return (nb + (jnp.arange(B, dtype=jnp.int32) * N)[:, None, None]
            ).reshape(B, -1)


def _pad_planes(pl3, width):
    if pl3.shape[2] == width:
        return pl3
    pad = width - pl3.shape[2]
    return jnp.pad(pl3, ((0, 0), (0, 0), (0, pad)))


def _transition_down(nb_x, local, W, b):
    feats = jax.nn.relu(_dense(jnp.concatenate([nb_x, local], -1), W, b))
    return jnp.max(feats, axis=2)


def kernel(x, coords, params):
    x = jnp.transpose(x, (0, 2, 1))
    h = jax.nn.relu(_dense(x, params['Win'], params['bin']))
    c0pl = (coords[:, :, 0], coords[:, :, 1], coords[:, :, 2])
    nb0 = _knn_sc(_dist(coords, coords), N_PTS, N_PTS)
    kf0, lc0 = _group_sc(h.reshape(B * N_PTS, -1), _global_idx(nb0, N_PTS),
                         c0pl, c0pl, N_PTS, N_PTS, h.shape[-1])
    h = _pt_block_pallas(h, kf0, lc0, params['pt0'])
    fps_idxs, fps_cpl = _fps_sc(coords)
    plns = [c0pl] + [tuple(p[:, i_, :] for i_ in range(3)) for p in fps_cpl]
    c4pl = tuple(jnp.take_along_axis(p, fps_idxs[3], axis=1)
                 for p in plns[3])
    plns.append(tuple(jnp.pad(p, ((0, 0), (0, 12))) for p in c4pl))
    sizes = [N_PTS] + NUM_POINTS
    for i in range(len(NUM_POINTS)):
        M, N = sizes[i + 1], sizes[i]
        qc = jnp.stack(plns[i + 1], -1)[:, :M]
        rc = jnp.stack(plns[i], -1)
        nb = _knn_sc(_dist(qc, rc), M, N)
        nb_x, local = _group_sc(h.reshape(B * N, -1), _global_idx(nb, N),
                                plns[i + 1], plns[i], M, N, h.shape[-1])
        h = _transition_down(nb_x, local,
                             params['td'][i]['W'], params['td'][i]['b'])
    h = jnp.mean(h, axis=1)
    h = jax.nn.relu(_dense(h, params['dec'][0]['W'], params['dec'][0]['b']))
    h = jax.nn.relu(_dense(h, params['dec'][1]['W'], params['dec'][1]['b']))
    return _dense(h, params['dec'][2]['W'], params['dec'][2]['b'])
